# baseline (speedup 1.0000x reference)
import functools

import jax
import jax.numpy as jnp
from jax import lax
from jax.experimental import pallas as pl
from jax.experimental.pallas import tpu as pltpu
from jax.experimental.pallas import tpu_sc as plsc

B, S, V = 32, 192, 8192
N = B * S
RB = 32

NC, NS, L = 2, 16, 16
NW = NC * NS
TPW = N // NW
GCH = 96


def _sc_gather_body(table, senf, noif, picked, xnoise,
                    sen_v, noi_v, rs_i, rn_i, out_s, out_n, sem):
    wid = lax.axis_index("s") * NC + lax.axis_index("c")
    base = wid * TPW
    pltpu.sync_copy(senf.at[pl.ds(base, TPW)], sen_v)
    pltpu.sync_copy(noif.at[pl.ds(base, TPW)], noi_v)
    for g in range(TPW // L):
        sl = pl.ds(g * L, L)
        toks = (base + g * L) * V + lax.iota(jnp.int32, L) * V
        rs_i[sl] = toks + sen_v[sl]
        rn_i[sl] = toks + noi_v[sl]
    cps = []
    for h in range(TPW // GCH):
        sl = pl.ds(h * GCH, GCH)
        cps.append(pltpu.async_copy(table.at[rs_i.at[sl]], out_s.at[sl], sem))
        cps.append(pltpu.async_copy(table.at[rn_i.at[sl]], out_n.at[sl], sem))
    for cp in cps:
        cp.wait()
    pltpu.sync_copy(out_s, picked.at[pl.ds(base, TPW)])
    pltpu.sync_copy(out_n, xnoise.at[pl.ds(base, TPW)])


def _sc_gather(logits2d, sen_flat, noi_flat):
    table = logits2d.reshape(N * V)
    mesh = plsc.VectorSubcoreMesh(core_axis_name="c", subcore_axis_name="s")
    f = pl.kernel(
        _sc_gather_body,
        mesh=mesh,
        out_type=[jax.ShapeDtypeStruct((N,), jnp.float32)] * 2,
        scratch_types=[
            pltpu.VMEM((TPW,), jnp.int32),
            pltpu.VMEM((TPW,), jnp.int32),
            pltpu.VMEM((TPW,), jnp.int32),
            pltpu.VMEM((TPW,), jnp.int32),
            pltpu.VMEM((TPW,), jnp.float32),
            pltpu.VMEM((TPW,), jnp.float32),
            pltpu.SemaphoreType.DMA,
        ],
    )
    return f(table, sen_flat, noi_flat)


def _reduce_body(x_ref, m_ref, s_ref):
    x = x_ref[...]
    m = jnp.max(x, axis=1, keepdims=True)
    e = jnp.exp(x - m)
    s = jnp.sum(e, axis=1, keepdims=True)
    m_ref[...] = m
    s_ref[...] = s


def _finish_body(m_ref, s_ref, p_ref, n_ref, sen_ref, noi_ref, msk_ref,
                 th_ref, thup_ref,
                 ratio_ref, e0_ref, e1_ref, sc_ref):
    m = m_ref[...]
    s = s_ref[...]
    p = p_ref[...]
    xn = n_ref[...]
    sen = sen_ref[...]
    noi = noi_ref[...]
    msk = msk_ref[...] != 0
    thresh = th_ref[0, 0]
    thup = thup_ref[0, 0]

    one = jnp.ones_like(m)
    zero = jnp.zeros_like(m)

    lse = m + jnp.log(s)
    loss = jnp.sum(jnp.where(msk, lse - p, zero))
    argeq = p == m
    acc = jnp.sum(jnp.where(msk & argeq, one, zero))

    col = jax.lax.broadcasted_iota(jnp.int32, m.shape, 1)
    inner = (col >= 1) & (col <= S - 2)

    ratio0 = jnp.exp(m - xn)
    e0raw = ratio0 > thup
    eraw = (ratio0 < thresh) & (~e0raw)
    china = (noi > 670) & (noi < 7992)
    err = eraw | (~china)
    e0out = (~e0raw) | (~china)

    ratio_ref[...] = jnp.where(err, one, ratio0)
    e0_ref[...] = e0out.astype(jnp.int32)
    e1_ref[...] = err.astype(jnp.int32)

    binl = noi == sen
    topeq = err | argeq

    def msum(b):
        return jnp.sum(jnp.where(b & inner, one, zero))

    tpd = (~binl) & (~err)
    tnd = (~binl) & err
    fpd = binl & (~err)
    TPD = msum(tpd)
    TND = msum(tnd)
    FPD = msum(fpd)
    tpc = tpd & topeq
    tnc = tnd | (tpd & (~topeq))
    TPC = msum(tpc)
    TNC = msum(tnc)
    FPC = FPD

    ione = jnp.ones_like(col)
    izero = jnp.zeros_like(col)
    bls = jnp.sum(jnp.where(inner & (~binl), ione, izero), axis=1, keepdims=True)
    lme = jnp.sum(jnp.where(inner & (binl != err), ione, izero), axis=1, keepdims=True)
    ntop = jnp.sum(jnp.where(inner & (~topeq), ione, izero), axis=1, keepdims=True)
    topsen = ntop == 0

    one_s = jnp.ones_like(bls, dtype=jnp.float32)
    zero_s = jnp.zeros_like(one_s)

    def ssum(b):
        return jnp.sum(jnp.where(b, one_s, zero_s))

    tpsd = (bls > 0) & (lme == 0)
    tnsd = (bls > 0) & (lme > 0)
    fpsd = (bls == 0) & (lme > 0)
    TPSD = ssum(tpsd)
    TNSD = ssum(tnsd)
    FPSD = ssum(fpsd)
    tpsc = tpsd & topsen
    tnsc = (bls > 0) & ((lme > 0) | ((lme == 0) & (~topsen)))
    TPSC = ssum(tpsc)
    TNSC = ssum(tnsc)
    FPSC = FPSD

    def prf(TP, TN, FP):
        eps = jnp.float32(1e-8)
        P = TP / (TP + FP + eps)
        R = TP / (TP + TN + eps)
        F = jnp.float32(2.0) * P * R / (P + R + eps)
        return P, R, F

    PD, RD, FD = prf(TPD, TND, FPD)
    PC, RC, FC = prf(TPC, TNC, FPC)
    PSD, RSD, FSD = prf(TPSD, TNSD, FPSD)
    PSC, RSC, FSC = prf(TPSC, TNSC, FPSC)

    vals = (loss, acc, TPD, TND, FPD, TPC, TNC, FPC,
            TPSD, TNSD, FPSD, TPSC, TNSC, FPSC,
            PD, RD, FD, PC, RC, FC, PSD, RSD, FSD, PSC, RSC, FSC)
    for k, v in enumerate(vals):
        sc_ref[0, k] = v


def kernel(sen, noise, logits, sequence_mask, sumls, pri, thresh, threshup):
    del pri
    x2 = logits.reshape(N, V)

    picked_flat, xn_flat = _sc_gather(x2, sen.reshape(N), noise.reshape(N))

    grid = N // RB
    m1, s1 = pl.pallas_call(
        _reduce_body,
        grid=(grid,),
        in_specs=[pl.BlockSpec((RB, V), lambda i: (i, 0))],
        out_specs=[
            pl.BlockSpec((RB, 1), lambda i: (i, 0)),
            pl.BlockSpec((RB, 1), lambda i: (i, 0)),
        ],
        out_shape=[jax.ShapeDtypeStruct((N, 1), jnp.float32)] * 2,
    )(x2)

    m2 = m1.reshape(B, S)
    s2 = s1.reshape(B, S)
    p2 = picked_flat.reshape(B, S)
    n2 = xn_flat.reshape(B, S)
    msk = sequence_mask.astype(jnp.int32)
    th = thresh.reshape(1, 1)
    thup = threshup.reshape(1, 1)

    ratio_f, e0_f, e1_f, scal = pl.pallas_call(
        _finish_body,
        in_specs=[pl.BlockSpec((B, S), lambda: (0, 0))] * 7
        + [pl.BlockSpec(memory_space=pltpu.SMEM)] * 2,
        out_specs=[
            pl.BlockSpec((B, S), lambda: (0, 0)),
            pl.BlockSpec((B, S), lambda: (0, 0)),
            pl.BlockSpec((B, S), lambda: (0, 0)),
            pl.BlockSpec(memory_space=pltpu.SMEM),
        ],
        out_shape=[
            jax.ShapeDtypeStruct((B, S), jnp.float32),
            jax.ShapeDtypeStruct((B, S), jnp.int32),
            jax.ShapeDtypeStruct((B, S), jnp.int32),
            jax.ShapeDtypeStruct((1, 32), jnp.float32),
        ],
    )(m2, s2, p2, n2, sen, noise, msk, th, thup)

    loss = scal[0, 0]
    acc = scal[0, 1]
    ratio = ratio_f[:, 1:S - 1]
    errtest0 = e0_f[:, 1:S - 1]
    errtest = e1_f[:, 1:S - 1]
    prf_scal = tuple(scal[0, k] for k in range(2, 26))
    return (loss, acc, sumls, ratio, errtest0, errtest) + prf_scal


# speedup vs baseline: 1.9392x; 1.9392x over previous
import functools

import jax
import jax.numpy as jnp
from jax import lax
from jax.experimental import pallas as pl
from jax.experimental.pallas import tpu as pltpu
from jax.experimental.pallas import tpu_sc as plsc

B, S, V = 32, 192, 8192
N = B * S
RB = 32

NC, NS, L = 2, 16, 16
NW = NC * NS
TPW = N // NW
GCH = 96


def _sc_gather_body(table, senf, noif, picked, xnoise,
                    sen_v, noi_v, rs_i, rn_i, out_s, out_n, sem):
    wid = lax.axis_index("s") * NC + lax.axis_index("c")
    base = wid * TPW
    pltpu.sync_copy(senf.at[pl.ds(base, TPW)], sen_v)
    pltpu.sync_copy(noif.at[pl.ds(base, TPW)], noi_v)
    for g in range(TPW // L):
        sl = pl.ds(g * L, L)
        toks = (base + g * L) * V + lax.iota(jnp.int32, L) * V
        rs_i[sl] = toks + sen_v[sl]
        rn_i[sl] = toks + noi_v[sl]
    cps = []
    for h in range(TPW // GCH):
        sl = pl.ds(h * GCH, GCH)
        cps.append(pltpu.async_copy(table.at[rs_i.at[sl]], out_s.at[sl], sem))
        cps.append(pltpu.async_copy(table.at[rn_i.at[sl]], out_n.at[sl], sem))
    for cp in cps:
        cp.wait()
    pltpu.sync_copy(out_s, picked.at[pl.ds(base, TPW)])
    pltpu.sync_copy(out_n, xnoise.at[pl.ds(base, TPW)])


def _sc_gather(logits2d, sen_flat, noi_flat):
    table = logits2d.reshape(N * V)
    mesh = plsc.VectorSubcoreMesh(core_axis_name="c", subcore_axis_name="s")
    f = pl.kernel(
        _sc_gather_body,
        mesh=mesh,
        out_type=[jax.ShapeDtypeStruct((N,), jnp.float32)] * 2,
        scratch_types=[
            pltpu.VMEM((TPW,), jnp.int32),
            pltpu.VMEM((TPW,), jnp.int32),
            pltpu.VMEM((TPW,), jnp.int32),
            pltpu.VMEM((TPW,), jnp.int32),
            pltpu.VMEM((TPW,), jnp.float32),
            pltpu.VMEM((TPW,), jnp.float32),
            pltpu.SemaphoreType.DMA,
        ],
    )
    return f(table, sen_flat, noi_flat)


NBUF = 4
NSTEPS = N // RB


def _reduce_body(x_hbm, sen_ref, noi_ref, m_ref, s_ref, p_ref, n_ref,
                 buf, sems):
    i = pl.program_id(0)

    def issue(blk):
        slot = jax.lax.rem(blk, NBUF)
        pltpu.make_async_copy(
            x_hbm.at[pl.ds(blk * RB, RB), :], buf.at[slot], sems.at[slot]
        ).start()

    @pl.when(i == 0)
    def _():
        for k in range(NBUF - 1):
            issue(k)

    @pl.when(i + NBUF - 1 < NSTEPS)
    def _():
        issue(i + NBUF - 1)

    slot = jax.lax.rem(i, NBUF)
    pltpu.make_async_copy(
        x_hbm.at[pl.ds(i * RB, RB), :], buf.at[slot], sems.at[slot]
    ).wait()

    x = buf[slot]
    sen = sen_ref[...]
    noi = noi_ref[...]
    m = jnp.max(x, axis=1, keepdims=True)
    e = jnp.exp(x - m)
    s = jnp.sum(e, axis=1, keepdims=True)
    ids = jax.lax.broadcasted_iota(jnp.int32, x.shape, 1)
    zero = jnp.zeros_like(x)
    p = jnp.sum(jnp.where(ids == sen, x, zero), axis=1, keepdims=True)
    n = jnp.sum(jnp.where(ids == noi, x, zero), axis=1, keepdims=True)
    m_ref[...] = m
    s_ref[...] = s
    p_ref[...] = p
    n_ref[...] = n


def _finish_body(m_ref, s_ref, p_ref, n_ref, sen_ref, noi_ref, msk_ref,
                 th_ref, thup_ref,
                 ratio_ref, e0_ref, e1_ref, sc_ref):
    m = m_ref[...]
    s = s_ref[...]
    p = p_ref[...]
    xn = n_ref[...]
    sen = sen_ref[...]
    noi = noi_ref[...]
    msk = msk_ref[...] != 0
    thresh = th_ref[0, 0]
    thup = thup_ref[0, 0]

    one = jnp.ones_like(m)
    zero = jnp.zeros_like(m)

    lse = m + jnp.log(s)
    loss = jnp.sum(jnp.where(msk, lse - p, zero))
    argeq = p == m
    acc = jnp.sum(jnp.where(msk & argeq, one, zero))

    col = jax.lax.broadcasted_iota(jnp.int32, m.shape, 1)
    inner = (col >= 1) & (col <= S - 2)

    ratio0 = jnp.exp(m - xn)
    e0raw = ratio0 > thup
    eraw = (ratio0 < thresh) & (~e0raw)
    china = (noi > 670) & (noi < 7992)
    err = eraw | (~china)
    e0out = (~e0raw) | (~china)

    ratio_ref[...] = jnp.where(err, one, ratio0)
    e0_ref[...] = e0out.astype(jnp.int32)
    e1_ref[...] = err.astype(jnp.int32)

    binl = noi == sen
    topeq = err | argeq

    def msum(b):
        return jnp.sum(jnp.where(b & inner, one, zero))

    tpd = (~binl) & (~err)
    tnd = (~binl) & err
    fpd = binl & (~err)
    TPD = msum(tpd)
    TND = msum(tnd)
    FPD = msum(fpd)
    tpc = tpd & topeq
    tnc = tnd | (tpd & (~topeq))
    TPC = msum(tpc)
    TNC = msum(tnc)
    FPC = FPD

    ione = jnp.ones_like(col)
    izero = jnp.zeros_like(col)
    bls = jnp.sum(jnp.where(inner & (~binl), ione, izero), axis=1, keepdims=True)
    lme = jnp.sum(jnp.where(inner & (binl != err), ione, izero), axis=1, keepdims=True)
    ntop = jnp.sum(jnp.where(inner & (~topeq), ione, izero), axis=1, keepdims=True)
    topsen = ntop == 0

    one_s = jnp.ones_like(bls, dtype=jnp.float32)
    zero_s = jnp.zeros_like(one_s)

    def ssum(b):
        return jnp.sum(jnp.where(b, one_s, zero_s))

    tpsd = (bls > 0) & (lme == 0)
    tnsd = (bls > 0) & (lme > 0)
    fpsd = (bls == 0) & (lme > 0)
    TPSD = ssum(tpsd)
    TNSD = ssum(tnsd)
    FPSD = ssum(fpsd)
    tpsc = tpsd & topsen
    tnsc = (bls > 0) & ((lme > 0) | ((lme == 0) & (~topsen)))
    TPSC = ssum(tpsc)
    TNSC = ssum(tnsc)
    FPSC = FPSD

    def prf(TP, TN, FP):
        eps = jnp.float32(1e-8)
        P = TP / (TP + FP + eps)
        R = TP / (TP + TN + eps)
        F = jnp.float32(2.0) * P * R / (P + R + eps)
        return P, R, F

    PD, RD, FD = prf(TPD, TND, FPD)
    PC, RC, FC = prf(TPC, TNC, FPC)
    PSD, RSD, FSD = prf(TPSD, TNSD, FPSD)
    PSC, RSC, FSC = prf(TPSC, TNSC, FPSC)

    vals = (loss, acc, TPD, TND, FPD, TPC, TNC, FPC,
            TPSD, TNSD, FPSD, TPSC, TNSC, FPSC,
            PD, RD, FD, PC, RC, FC, PSD, RSD, FSD, PSC, RSC, FSC)
    for k, v in enumerate(vals):
        sc_ref[0, k] = v


def kernel(sen, noise, logits, sequence_mask, sumls, pri, thresh, threshup):
    del pri
    x2 = logits.reshape(N, V)

    grid = N // RB
    m1, s1, p1, n1 = pl.pallas_call(
        _reduce_body,
        grid=(grid,),
        in_specs=[
            pl.BlockSpec(memory_space=pl.ANY),
            pl.BlockSpec((RB, 1), lambda i: (i, 0)),
            pl.BlockSpec((RB, 1), lambda i: (i, 0)),
        ],
        out_specs=[
            pl.BlockSpec((RB, 1), lambda i: (i, 0)),
            pl.BlockSpec((RB, 1), lambda i: (i, 0)),
            pl.BlockSpec((RB, 1), lambda i: (i, 0)),
            pl.BlockSpec((RB, 1), lambda i: (i, 0)),
        ],
        out_shape=[jax.ShapeDtypeStruct((N, 1), jnp.float32)] * 4,
        scratch_shapes=[
            pltpu.VMEM((NBUF, RB, V), jnp.float32),
            pltpu.SemaphoreType.DMA((NBUF,)),
        ],
    )(x2, sen.reshape(N, 1), noise.reshape(N, 1))

    m2 = m1.reshape(B, S)
    s2 = s1.reshape(B, S)
    p2 = p1.reshape(B, S)
    n2 = n1.reshape(B, S)
    msk = sequence_mask.astype(jnp.int32)
    th = thresh.reshape(1, 1)
    thup = threshup.reshape(1, 1)

    ratio_f, e0_f, e1_f, scal = pl.pallas_call(
        _finish_body,
        in_specs=[pl.BlockSpec((B, S), lambda: (0, 0))] * 7
        + [pl.BlockSpec(memory_space=pltpu.SMEM)] * 2,
        out_specs=[
            pl.BlockSpec((B, S), lambda: (0, 0)),
            pl.BlockSpec((B, S), lambda: (0, 0)),
            pl.BlockSpec((B, S), lambda: (0, 0)),
            pl.BlockSpec(memory_space=pltpu.SMEM),
        ],
        out_shape=[
            jax.ShapeDtypeStruct((B, S), jnp.float32),
            jax.ShapeDtypeStruct((B, S), jnp.int32),
            jax.ShapeDtypeStruct((B, S), jnp.int32),
            jax.ShapeDtypeStruct((1, 32), jnp.float32),
        ],
    )(m2, s2, p2, n2, sen, noise, msk, th, thup)

    loss = scal[0, 0]
    acc = scal[0, 1]
    ratio = ratio_f[:, 1:S - 1]
    errtest0 = e0_f[:, 1:S - 1]
    errtest = e1_f[:, 1:S - 1]
    prf_scal = tuple(scal[0, k] for k in range(2, 26))
    return (loss, acc, sumls, ratio, errtest0, errtest) + prf_scal
